# baseline (speedup 1.0000x reference)
import jax
import jax.numpy as jnp
from jax.experimental import pallas as pl
from jax.experimental.pallas import tpu as pltpu


def _gcn_body(adjA_ref, adjB_ref, v_ref, w1_ref, wout_ref, bout_ref,
              out_ref, support_ref, x_ref):
    r = pl.program_id(0)
    R = x_ref.shape[0] // 2
    BR = x_ref.shape[1]

    @pl.when(r == 0)
    def _init():
        support_ref[...] = jnp.dot(
            v_ref[...], w1_ref[...], preferred_element_type=jnp.float32
        )

    hTA = jax.lax.dot_general(
        support_ref[...], adjA_ref[...],
        dimension_numbers=(((0,), (1,)), ((), ())),
        preferred_element_type=jnp.float32,
    )
    hTB = jax.lax.dot_general(
        support_ref[...], adjB_ref[...],
        dimension_numbers=(((0,), (1,)), ((), ())),
        preferred_element_type=jnp.float32,
    )
    x_ref[pl.ds(r, 1), :] = jnp.sum(jax.nn.relu(hTA), axis=0, keepdims=True)
    x_ref[pl.ds(R + r, 1), :] = jnp.sum(jax.nn.relu(hTB), axis=0, keepdims=True)

    @pl.when(r == R - 1)
    def _fin():
        acc = bout_ref[...]
        for rr in range(2 * R):
            acc = acc + jnp.dot(
                x_ref[rr:rr + 1, :], wout_ref[rr * BR:(rr + 1) * BR, :],
                preferred_element_type=jnp.float32,
            )
        out_ref[...] = acc


def kernel(v, adj, W1, W_out, b_out):
    B, N, F = v.shape
    L = W_out.shape[1]
    H = W1.shape[1]

    v2 = v.reshape(N, F)
    adj2 = adj.reshape(N, N)
    bout2 = b_out.reshape(1, L)

    BR = 200
    NB = N // (2 * BR)
    grid = (NB,)

    out = pl.pallas_call(
        _gcn_body,
        grid=grid,
        in_specs=[
            pl.BlockSpec((BR, N), lambda r: (r, 0)),
            pl.BlockSpec((BR, N), lambda r: (NB + r, 0)),
            pl.BlockSpec((N, F), lambda r: (0, 0)),
            pl.BlockSpec((F, H), lambda r: (0, 0)),
            pl.BlockSpec((N, L), lambda r: (0, 0)),
            pl.BlockSpec((1, L), lambda r: (0, 0)),
        ],
        compiler_params=pltpu.CompilerParams(
            fuse_transposed_lhs_in_matmul=True,
        ),
        out_specs=pl.BlockSpec((1, L), lambda r: (0, 0)),
        out_shape=jax.ShapeDtypeStruct((1, L), jnp.float32),
        scratch_shapes=[
            pltpu.VMEM((N, H), jnp.float32),
            pltpu.VMEM((N // BR, BR), jnp.float32),
        ],
    )(adj2, adj2, v2, W1, W_out, bout2)

    return out.reshape(B, L)


# speedup vs baseline: 1.0142x; 1.0142x over previous
import jax
import jax.numpy as jnp
from jax.experimental import pallas as pl
from jax.experimental.pallas import tpu as pltpu


def _gcn_body(adjA_ref, adjB_ref, v_ref, w1_ref, wout_ref, bout_ref,
              out_ref, support_ref, x_ref):
    r = pl.program_id(0)
    R = x_ref.shape[0] // 2
    BR = x_ref.shape[1]

    @pl.when(r == 0)
    def _init():
        support_ref[...] = jnp.dot(
            v_ref[...], w1_ref[...], preferred_element_type=jnp.float32
        )

    hTA = jax.lax.dot_general(
        support_ref[...], adjA_ref[...],
        dimension_numbers=(((0,), (1,)), ((), ())),
        preferred_element_type=jnp.float32,
    )
    hTB = jax.lax.dot_general(
        support_ref[...], adjB_ref[...],
        dimension_numbers=(((0,), (1,)), ((), ())),
        preferred_element_type=jnp.float32,
    )
    x_ref[pl.ds(r, 1), :] = jnp.sum(jax.nn.relu(hTA), axis=0, keepdims=True)
    x_ref[pl.ds(R + r, 1), :] = jnp.sum(jax.nn.relu(hTB), axis=0, keepdims=True)

    @pl.when(r == R - 1)
    def _fin():
        acc = bout_ref[...]
        for rr in range(2 * R):
            acc = acc + jnp.dot(
                x_ref[rr:rr + 1, :], wout_ref[rr * BR:(rr + 1) * BR, :],
                preferred_element_type=jnp.float32,
            )
        out_ref[...] = acc


def kernel(v, adj, W1, W_out, b_out):
    B, N, F = v.shape
    L = W_out.shape[1]
    H = W1.shape[1]

    v2 = v.reshape(N, F)
    adj2 = adj.reshape(N, N)
    bout2 = b_out.reshape(1, L)

    BR = 200
    NB = N // (2 * BR)
    grid = (NB,)

    out = pl.pallas_call(
        _gcn_body,
        grid=grid,
        in_specs=[
            pl.BlockSpec((BR, N), lambda r: (r, 0)),
            pl.BlockSpec((BR, N), lambda r: (NB + r, 0)),
            pl.BlockSpec((N, F), lambda r: (0, 0)),
            pl.BlockSpec((F, H), lambda r: (0, 0)),
            pl.BlockSpec((N, L), lambda r: (0, 0)),
            pl.BlockSpec((1, L), lambda r: (0, 0)),
        ],
        out_specs=pl.BlockSpec((1, L), lambda r: (0, 0)),
        out_shape=jax.ShapeDtypeStruct((1, L), jnp.float32),
        scratch_shapes=[
            pltpu.VMEM((N, H), jnp.float32),
            pltpu.VMEM((N // BR, BR), jnp.float32),
        ],
    )(adj2, adj2, v2, W1, W_out, bout2)

    return out.reshape(B, L)
